# baseline (speedup 1.0000x reference)
import jax
import jax.numpy as jnp
from jax.experimental import pallas as pl
from jax.experimental.pallas import tpu as pltpu

_N = 21
_W = 262144


def _iou_kernel(p_ref, t_ref, out_ref, acc_ref):
    bi = pl.program_id(0)
    ji = pl.program_id(1)

    @pl.when((bi == 0) & (ji == 0))
    def _init():
        acc_ref[...] = jnp.zeros_like(acc_ref)

    x = p_ref[0]
    acc_ref[0:1, 0:1] += jnp.sum(x).reshape(1, 1)

    @pl.when((bi == pl.num_programs(0) - 1) & (ji == pl.num_programs(1) - 1))
    def _fin():
        out_ref[...] = acc_ref[0:1, 0:1]


def kernel(preds, targets, mat):
    batch, n, hh, ww = preds.shape
    pix = hh * ww
    p = preds.reshape(batch, n, pix)
    t = targets.reshape(batch, 1, pix)
    nb = pix // _W
    out = pl.pallas_call(
        _iou_kernel,
        grid=(batch, nb),
        in_specs=[
            pl.BlockSpec((1, n, _W), lambda b, j: (b, 0, j)),
            pl.BlockSpec((1, 1, _W), lambda b, j: (b, 0, j)),
        ],
        out_specs=pl.BlockSpec((1, 1), lambda b, j: (0, 0)),
        out_shape=jax.ShapeDtypeStruct((1, 1), jnp.float32),
        scratch_shapes=[pltpu.VMEM((_N, _N), jnp.float32)],
    )(p, t)
    return out[0, 0]


# speedup vs baseline: 1.0071x; 1.0071x over previous
import jax
import jax.numpy as jnp
from jax.experimental import pallas as pl
from jax.experimental.pallas import tpu as pltpu

_N = 21
_W = 131072


def _cm_kernel(p_ref, t_ref, out_ref):
    ji = pl.program_id(1)

    x = p_ref[0]
    t = t_ref[0]
    iota = jax.lax.broadcasted_iota(jnp.int32, (_N, 1), 0)
    maxv = jnp.max(x, axis=0, keepdims=True)
    b_oh = (x == maxv).astype(jnp.bfloat16)
    a_oh = (t == iota).astype(jnp.bfloat16)
    c = jax.lax.dot_general(
        a_oh, b_oh, (((1,), (1,)), ((), ())),
        preferred_element_type=jnp.float32)

    @pl.when(ji == 0)
    def _first():
        out_ref[0] = c

    @pl.when(ji != 0)
    def _rest():
        out_ref[0] += c


def _iou_eval_kernel(m_ref, out_ref):
    h = jnp.sum(m_ref[...], axis=0)
    r = jax.lax.broadcasted_iota(jnp.int32, (_N, _N), 0)
    cidx = jax.lax.broadcasted_iota(jnp.int32, (_N, _N), 1)
    eye = (r == cidx).astype(jnp.float32)
    ones = jnp.ones((_N, 1), jnp.float32)
    diag = jax.lax.dot_general(
        h * eye, ones, (((1,), (0,)), ((), ())),
        preferred_element_type=jnp.float32)
    rows = jax.lax.dot_general(
        h, ones, (((1,), (0,)), ((), ())),
        preferred_element_type=jnp.float32)
    cols = jax.lax.dot_general(
        h, ones, (((0,), (0,)), ((), ())),
        preferred_element_type=jnp.float32)
    iou = diag / (rows + cols - diag)
    out_ref[...] = (jnp.sum(iou) / _N).reshape(1, 1)


def kernel(preds, targets, mat):
    batch, n, hh, ww = preds.shape
    pix = hh * ww
    p = preds.reshape(batch, n, pix)
    t = targets.reshape(batch, 1, pix)
    nb = pix // _W
    mats = pl.pallas_call(
        _cm_kernel,
        grid=(batch, nb),
        in_specs=[
            pl.BlockSpec((1, n, _W), lambda b, j: (b, 0, j)),
            pl.BlockSpec((1, 1, _W), lambda b, j: (b, 0, j)),
        ],
        out_specs=pl.BlockSpec((1, n, n), lambda b, j: (b, 0, 0)),
        out_shape=jax.ShapeDtypeStruct((batch, n, n), jnp.float32),
        compiler_params=pltpu.CompilerParams(
            dimension_semantics=("parallel", "arbitrary")),
    )(p, t)
    out = pl.pallas_call(
        _iou_eval_kernel,
        out_shape=jax.ShapeDtypeStruct((1, 1), jnp.float32),
    )(mats)
    return out[0, 0]


# speedup vs baseline: 1.0073x; 1.0002x over previous
import jax
import jax.numpy as jnp
from jax.experimental import pallas as pl
from jax.experimental.pallas import tpu as pltpu

_N = 21
_W = 32768


def _cm_kernel(p0_ref, p1_ref, p2_ref, p3_ref, t_ref, out_ref):
    ji = pl.program_id(1)

    iota = jax.lax.broadcasted_iota(jnp.int32, (_N, 1), 0)
    c = jnp.zeros((_N, _N), jnp.float32)
    for k, p_ref in enumerate((p0_ref, p1_ref, p2_ref, p3_ref)):
        x = p_ref[0]
        t = t_ref[0, :, k * _W:(k + 1) * _W]
        maxv = jnp.max(x, axis=0, keepdims=True)
        b_oh = (x == maxv).astype(jnp.bfloat16)
        a_oh = (t == iota).astype(jnp.bfloat16)
        c = c + jax.lax.dot_general(
            a_oh, b_oh, (((1,), (1,)), ((), ())),
            preferred_element_type=jnp.float32)

    @pl.when(ji == 0)
    def _first():
        out_ref[0] = c

    @pl.when(ji != 0)
    def _rest():
        out_ref[0] += c


def _iou_eval_kernel(m_ref, out_ref):
    h = jnp.sum(m_ref[...], axis=0)
    r = jax.lax.broadcasted_iota(jnp.int32, (_N, _N), 0)
    cidx = jax.lax.broadcasted_iota(jnp.int32, (_N, _N), 1)
    eye = (r == cidx).astype(jnp.float32)
    ones = jnp.ones((_N, 1), jnp.float32)
    diag = jax.lax.dot_general(
        h * eye, ones, (((1,), (0,)), ((), ())),
        preferred_element_type=jnp.float32)
    rows = jax.lax.dot_general(
        h, ones, (((1,), (0,)), ((), ())),
        preferred_element_type=jnp.float32)
    cols = jax.lax.dot_general(
        h, ones, (((0,), (0,)), ((), ())),
        preferred_element_type=jnp.float32)
    iou = diag / (rows + cols - diag)
    out_ref[...] = (jnp.sum(iou) / _N).reshape(1, 1)


def kernel(preds, targets, mat):
    batch, n, hh, ww = preds.shape
    pix = hh * ww
    p = preds.reshape(batch, n, pix)
    t = targets.reshape(batch, 1, pix)
    nb = pix // (4 * _W)
    pspec = lambda k: pl.BlockSpec(
        (1, n, _W), lambda b, j, k=k: (b, 0, 4 * j + k))
    mats = pl.pallas_call(
        _cm_kernel,
        grid=(batch, nb),
        in_specs=[
            pspec(0), pspec(1), pspec(2), pspec(3),
            pl.BlockSpec((1, 1, 4 * _W), lambda b, j: (b, 0, j)),
        ],
        out_specs=pl.BlockSpec((1, n, n), lambda b, j: (b, 0, 0)),
        out_shape=jax.ShapeDtypeStruct((batch, n, n), jnp.float32),
        compiler_params=pltpu.CompilerParams(
            dimension_semantics=("parallel", "arbitrary")),
    )(p, p, p, p, t)
    out = pl.pallas_call(
        _iou_eval_kernel,
        out_shape=jax.ShapeDtypeStruct((1, 1), jnp.float32),
    )(mats)
    return out[0, 0]
